# baseline (speedup 1.0000x reference)
import functools

import numpy as np

import jax
import jax.numpy as jnp
from jax import lax
from jax.experimental import pallas as pl
from jax.experimental.pallas import tpu as pltpu
from jax.experimental.pallas import tpu_sc as plsc

_MASK_RATE = 0.1


def _rotl32(x, d):
    return ((x << np.uint32(d)) | (x >> np.uint32(32 - d))).astype(np.uint32)


def _threefry2x32(k1, k2, x0, x1):
    x0 = np.asarray(x0, np.uint32).copy()
    x1 = np.asarray(x1, np.uint32).copy()
    ks = [np.uint32(k1), np.uint32(k2),
          np.uint32(np.uint32(k1) ^ np.uint32(k2) ^ np.uint32(0x1BD11BDA))]
    rot = [[13, 15, 26, 6], [17, 29, 16, 24]]
    x0 = (x0 + ks[0]).astype(np.uint32)
    x1 = (x1 + ks[1]).astype(np.uint32)
    for i in range(5):
        for r in rot[i % 2]:
            x0 = (x0 + x1).astype(np.uint32)
            x1 = _rotl32(x1, r)
            x1 = (x1 ^ x0).astype(np.uint32)
        x0 = (x0 + ks[(i + 1) % 3]).astype(np.uint32)
        x1 = (x1 + ks[(i + 2) % 3] + np.uint32(i + 1)).astype(np.uint32)
    return x0, x1


def _mask_starts(batch_size: int, T: int):
    len_mask = int(round(T * _MASK_RATE))
    seed = 42
    k1 = np.uint32((seed >> 32) & 0xFFFFFFFF)
    k2 = np.uint32(seed & 0xFFFFFFFF)
    b1, b2 = _threefry2x32(k1, k2, np.zeros(2, np.uint32),
                           np.arange(2, dtype=np.uint32))
    zeros_n = np.zeros(batch_size, np.uint32)
    iota_n = np.arange(batch_size, dtype=np.uint32)
    h1, h2 = _threefry2x32(b1[0], b2[0], zeros_n, iota_n)
    l1, l2 = _threefry2x32(b1[1], b2[1], zeros_n, iota_n)
    higher = (h1 ^ h2).astype(np.uint32)
    lower = (l1 ^ l2).astype(np.uint32)
    span = np.uint32(T - len_mask)
    mult = np.uint32(np.uint32(2 ** 16) % span)
    mult = np.uint32((mult * mult) % span)
    off = ((higher % span) * mult + (lower % span)).astype(np.uint32) % span
    return [int(v) for v in off], len_mask


def kernel(input):
    B, C, T, D = input.shape
    starts, L = _mask_starts(B, T)
    NE = B * C * T * D
    x1 = input.reshape(NE)
    zeros = jnp.zeros((L * D,), input.dtype)

    info = plsc.get_sparse_core_info()
    NC, NS = info.num_cores, info.num_subcores
    NW = NC * NS
    pairs = B * C
    per_w = pairs // NW
    span = per_w * T * D
    CHR = 256
    CH = CHR * D
    rows_w = per_w * T
    plan = []
    r = 0
    while r < rows_w:
        n = min(CHR, rows_w - r)
        plan.append((r * D, n * D))
        r += n
    NCHUNK = len(plan)
    NBUF = 3
    last_chunk = [((j + 1) * T - 1) // CHR for j in range(per_w)]
    band_at = {}
    for j in range(per_w):
        band_at.setdefault(last_chunk[j] + NBUF, []).append(j)

    mesh = plsc.VectorSubcoreMesh(core_axis_name="c", subcore_axis_name="s")

    @functools.partial(
        pl.kernel,
        mesh=mesh,
        out_type=jax.ShapeDtypeStruct((NE,), input.dtype),
        scratch_types=[
            pltpu.VMEM_SHARED((16 * NBUF * CH,), jnp.float32),
            pltpu.VMEM((L * D,), jnp.float32),
            pltpu.SemaphoreType.DMA,
            pltpu.SemaphoreType.DMA,
            pltpu.SemaphoreType.DMA,
            pltpu.SemaphoreType.DMA,
            pltpu.SemaphoreType.DMA,
            pltpu.SemaphoreType.DMA,
            pltpu.SemaphoreType.DMA,
            pltpu.SemaphoreType.DMA,
        ],
    )
    def sc_fn(x_hbm, z_hbm, out_hbm, shared, zbuf,
              in_sem0, in_sem1, in_sem2, out_sem0, out_sem1, out_sem2,
              z_sem, band_sem):
        sid = lax.axis_index("s")
        bufs = [shared.at[pl.ds((sid * NBUF + b) * CH, CH)]
                for b in range(NBUF)]
        in_sems = [in_sem0, in_sem1, in_sem2]
        out_sems = [out_sem0, out_sem1, out_sem2]
        wid = lax.axis_index("s") * NC + lax.axis_index("c")
        base = wid * span
        z_h = pltpu.async_copy(z_hbm, zbuf, z_sem)

        bi = wid // (C // per_w)
        s = jnp.int32(starts[0])
        for i in range(1, B):
            s = jnp.where(bi == i, jnp.int32(starts[i]), s)

        def issue_band(j):
            e0 = base + j * T * D + s * D
            return pltpu.async_copy(zbuf, out_hbm.at[pl.ds(e0, L * D)],
                                    band_sem)

        in_h = [None] * NBUF
        out_h = [None] * NBUF
        band_h = []
        z_waited = False
        for k in range(NCHUNK + 1):
            if k < NCHUNK:
                b = k % NBUF
                if out_h[b] is not None:
                    out_h[b].wait()
                for j in band_at.get(k, ()):
                    if not z_waited:
                        z_h.wait()
                        z_waited = True
                    band_h.append(issue_band(j))
                off, sz = plan[k]
                in_h[b] = pltpu.async_copy(
                    x_hbm.at[pl.ds(base + off, sz)],
                    bufs[b] if sz == CH else shared.at[pl.ds(
                        (sid * NBUF + b) * CH, sz)],
                    in_sems[b])
            if k >= 1:
                pb = (k - 1) % NBUF
                in_h[pb].wait()
                poff, psz = plan[k - 1]
                out_h[pb] = pltpu.async_copy(
                    bufs[pb] if psz == CH else shared.at[pl.ds(
                        (sid * NBUF + pb) * CH, psz)],
                    out_hbm.at[pl.ds(base + poff, psz)],
                    out_sems[pb])
        for b in range(NBUF):
            if out_h[b] is not None:
                out_h[b].wait()
        for kk, js in band_at.items():
            if kk >= NCHUNK:
                for j in js:
                    if not z_waited:
                        z_h.wait()
                        z_waited = True
                    band_h.append(issue_band(j))
        for h in band_h:
            h.wait()

    out1 = sc_fn(x1, zeros)
    return out1.reshape(B, C, T, D)


# speedup vs baseline: 1.0169x; 1.0169x over previous
import functools

import numpy as np

import jax
import jax.numpy as jnp
from jax import lax
from jax.experimental import pallas as pl
from jax.experimental.pallas import tpu as pltpu
from jax.experimental.pallas import tpu_sc as plsc

_MASK_RATE = 0.1


def _rotl32(x, d):
    return ((x << np.uint32(d)) | (x >> np.uint32(32 - d))).astype(np.uint32)


def _threefry2x32(k1, k2, x0, x1):
    x0 = np.asarray(x0, np.uint32).copy()
    x1 = np.asarray(x1, np.uint32).copy()
    ks = [np.uint32(k1), np.uint32(k2),
          np.uint32(np.uint32(k1) ^ np.uint32(k2) ^ np.uint32(0x1BD11BDA))]
    rot = [[13, 15, 26, 6], [17, 29, 16, 24]]
    x0 = (x0 + ks[0]).astype(np.uint32)
    x1 = (x1 + ks[1]).astype(np.uint32)
    for i in range(5):
        for r in rot[i % 2]:
            x0 = (x0 + x1).astype(np.uint32)
            x1 = _rotl32(x1, r)
            x1 = (x1 ^ x0).astype(np.uint32)
        x0 = (x0 + ks[(i + 1) % 3]).astype(np.uint32)
        x1 = (x1 + ks[(i + 2) % 3] + np.uint32(i + 1)).astype(np.uint32)
    return x0, x1


def _mask_starts(batch_size: int, T: int):
    len_mask = int(round(T * _MASK_RATE))
    seed = 42
    k1 = np.uint32((seed >> 32) & 0xFFFFFFFF)
    k2 = np.uint32(seed & 0xFFFFFFFF)
    b1, b2 = _threefry2x32(k1, k2, np.zeros(2, np.uint32),
                           np.arange(2, dtype=np.uint32))
    zeros_n = np.zeros(batch_size, np.uint32)
    iota_n = np.arange(batch_size, dtype=np.uint32)
    h1, h2 = _threefry2x32(b1[0], b2[0], zeros_n, iota_n)
    l1, l2 = _threefry2x32(b1[1], b2[1], zeros_n, iota_n)
    higher = (h1 ^ h2).astype(np.uint32)
    lower = (l1 ^ l2).astype(np.uint32)
    span = np.uint32(T - len_mask)
    mult = np.uint32(np.uint32(2 ** 16) % span)
    mult = np.uint32((mult * mult) % span)
    off = ((higher % span) * mult + (lower % span)).astype(np.uint32) % span
    return [int(v) for v in off], len_mask


def kernel(input):
    B, C, T, D = input.shape
    starts, L = _mask_starts(B, T)
    NE = B * C * T * D
    x1 = input.reshape(NE)
    zeros = jnp.zeros((L * D,), input.dtype)

    info = plsc.get_sparse_core_info()
    NC, NS = info.num_cores, info.num_subcores
    NW = NC * NS
    pairs = B * C
    per_w = pairs // NW
    span = per_w * T * D
    CHR = 256
    CH = CHR * D
    rows_w = per_w * T
    plan = []
    r = 0
    while r < rows_w:
        n = min(CHR, rows_w - r)
        plan.append((r * D, n * D))
        r += n
    NCHUNK = len(plan)
    NBUF = 3
    band_at = {NCHUNK: list(range(per_w))}

    mesh = plsc.VectorSubcoreMesh(core_axis_name="c", subcore_axis_name="s")

    @functools.partial(
        pl.kernel,
        mesh=mesh,
        out_type=jax.ShapeDtypeStruct((NE,), input.dtype),
        scratch_types=[
            pltpu.VMEM_SHARED((16 * NBUF * CH,), jnp.float32),
            pltpu.VMEM((L * D,), jnp.float32),
            pltpu.SemaphoreType.DMA,
            pltpu.SemaphoreType.DMA,
            pltpu.SemaphoreType.DMA,
            pltpu.SemaphoreType.DMA,
            pltpu.SemaphoreType.DMA,
            pltpu.SemaphoreType.DMA,
            pltpu.SemaphoreType.DMA,
            pltpu.SemaphoreType.DMA,
        ],
    )
    def sc_fn(x_hbm, z_hbm, out_hbm, shared, zbuf,
              in_sem0, in_sem1, in_sem2, out_sem0, out_sem1, out_sem2,
              z_sem, band_sem):
        sid = lax.axis_index("s")
        bufs = [shared.at[pl.ds((sid * NBUF + b) * CH, CH)]
                for b in range(NBUF)]
        in_sems = [in_sem0, in_sem1, in_sem2]
        out_sems = [out_sem0, out_sem1, out_sem2]
        wid = lax.axis_index("s") * NC + lax.axis_index("c")
        base = wid * span
        z_h = pltpu.async_copy(z_hbm, zbuf, z_sem)

        bi = wid // (C // per_w)
        s = jnp.int32(starts[0])
        for i in range(1, B):
            s = jnp.where(bi == i, jnp.int32(starts[i]), s)

        def issue_band(j):
            e0 = base + j * T * D + s * D
            return pltpu.async_copy(zbuf, out_hbm.at[pl.ds(e0, L * D)],
                                    band_sem)

        in_h = [None] * NBUF
        out_h = [None] * NBUF
        band_h = []
        z_waited = False
        for k in range(NCHUNK + 1):
            if k < NCHUNK:
                b = k % NBUF
                if out_h[b] is not None:
                    out_h[b].wait()
                for j in band_at.get(k, ()):
                    if not z_waited:
                        z_h.wait()
                        z_waited = True
                    band_h.append(issue_band(j))
                off, sz = plan[k]
                in_h[b] = pltpu.async_copy(
                    x_hbm.at[pl.ds(base + off, sz)],
                    bufs[b] if sz == CH else shared.at[pl.ds(
                        (sid * NBUF + b) * CH, sz)],
                    in_sems[b])
            if k >= 1:
                pb = (k - 1) % NBUF
                in_h[pb].wait()
                poff, psz = plan[k - 1]
                out_h[pb] = pltpu.async_copy(
                    bufs[pb] if psz == CH else shared.at[pl.ds(
                        (sid * NBUF + pb) * CH, psz)],
                    out_hbm.at[pl.ds(base + poff, psz)],
                    out_sems[pb])
        for b in range(NBUF):
            if out_h[b] is not None:
                out_h[b].wait()
        for kk, js in band_at.items():
            if kk >= NCHUNK:
                for j in js:
                    if not z_waited:
                        z_h.wait()
                        z_waited = True
                    band_h.append(issue_band(j))
        for h in band_h:
            h.wait()

    out1 = sc_fn(x1, zeros)
    return out1.reshape(B, C, T, D)
